# Initial kernel scaffold; baseline (speedup 1.0000x reference)
import jax
import jax.numpy as jnp
from jax.experimental import pallas as pl


def kernel(t_in, y_in, inds_r1_1st, inds_p1_1st, inds_p2_1st, inds_r1_2nd, inds_r2_2nd, inds_p1_2nd, inds_p2_2nd, alpha_1st, beta_1st, gamma_1st, alpha_2nd, beta_2nd, gamma_2nd, T0, den0):
    raise NotImplementedError("write your pallas kernel here")


# speedup vs baseline: 2.3540x; 2.3540x over previous
import jax
import jax.numpy as jnp
from jax import lax
from jax.experimental import pallas as pl
from jax.experimental.pallas import tpu as pltpu
from jax.experimental.pallas import tpu_sc as plsc

B = 128
S = 512
R1 = 20000
R2 = 80000

NCORE = 2
NSUB = 16
BH = B // NCORE

R1G = R1 // NSUB
R2G = R2 // NSUB
R1GP = ((R1G + 15) // 16) * 16
R2GP = ((R2G + 15) // 16) * 16
NG1 = R1GP // 16
NG2 = R2GP // 16


def _pad_groups(x, rg, rgp, nsub):
    x = x.reshape(nsub, rg)
    x = jnp.pad(x, ((0, 0), (0, rgp - rg)))
    return x.reshape(nsub * rgp)


def _sc_body(y_h, ltb_h, ntb_h,
             i1r1_h, i1p1_h, i1p2_h, a1_h, b1_h, g1_h,
             i2r1_h, i2r2_h, i2p1_h, i2p2_h, a2_h, b2_h, g2_h,
             out_h,
             y_v, dy_v, ltb_v, ntb_v,
             i1r1_v, i1p1_v, i1p2_v, a1_v, b1_v, g1_v,
             i2r1_v, i2r2_v, i2p1_v, i2p2_v, a2_v, b2_v, g2_v):
    c = lax.axis_index("c")
    s = lax.axis_index("s")

    pltpu.sync_copy(y_h.at[pl.ds(c * (BH * S), BH * S)], y_v)
    pltpu.sync_copy(ltb_h.at[pl.ds(c * (BH * 16), BH * 16)], ltb_v)
    pltpu.sync_copy(ntb_h.at[pl.ds(c * (BH * 16), BH * 16)], ntb_v)
    pltpu.sync_copy(i1r1_h.at[pl.ds(s * R1GP, R1GP)], i1r1_v)
    pltpu.sync_copy(i1p1_h.at[pl.ds(s * R1GP, R1GP)], i1p1_v)
    pltpu.sync_copy(i1p2_h.at[pl.ds(s * R1GP, R1GP)], i1p2_v)
    pltpu.sync_copy(a1_h.at[pl.ds(s * R1GP, R1GP)], a1_v)
    pltpu.sync_copy(b1_h.at[pl.ds(s * R1GP, R1GP)], b1_v)
    pltpu.sync_copy(g1_h.at[pl.ds(s * R1GP, R1GP)], g1_v)
    pltpu.sync_copy(i2r1_h.at[pl.ds(s * R2GP, R2GP)], i2r1_v)
    pltpu.sync_copy(i2r2_h.at[pl.ds(s * R2GP, R2GP)], i2r2_v)
    pltpu.sync_copy(i2p1_h.at[pl.ds(s * R2GP, R2GP)], i2p1_v)
    pltpu.sync_copy(i2p2_h.at[pl.ds(s * R2GP, R2GP)], i2p2_v)
    pltpu.sync_copy(a2_h.at[pl.ds(s * R2GP, R2GP)], a2_v)
    pltpu.sync_copy(b2_h.at[pl.ds(s * R2GP, R2GP)], b2_v)
    pltpu.sync_copy(g2_h.at[pl.ds(s * R2GP, R2GP)], g2_v)

    zeros16 = jnp.zeros((16,), jnp.float32)

    def zero_row(i, carry):
        for j in range(8):
            dy_v[pl.ds(i * 128 + j * 16, 16)] = zeros16
        return carry

    lax.fori_loop(0, BH * S // 128, zero_row, 0)

    def group2(ig, carry):
        base = ig * 16
        ir1 = i2r1_v[pl.ds(base, 16)]
        ir2 = i2r2_v[pl.ds(base, 16)]
        ip1 = i2p1_v[pl.ds(base, 16)]
        ip2 = i2p2_v[pl.ds(base, 16)]
        av = a2_v[pl.ds(base, 16)]
        bv = b2_v[pl.ds(base, 16)]
        gv = g2_v[pl.ds(base, 16)]

        def brow(b, carry2):
            ltv = ltb_v[pl.ds(b * 16, 16)]
            ntv = ntb_v[pl.ds(b * 16, 16)]
            bs = jnp.full((16,), b * S, dtype=jnp.int32)
            fr1 = bs + ir1
            fr2 = bs + ir2
            y1 = plsc.load_gather(y_v, [fr1])
            y2 = plsc.load_gather(y_v, [fr2])
            k = av * jnp.exp(bv * ltv + gv * ntv)
            t = k * y1 * y2
            nt = -t
            plsc.addupdate_scatter(dy_v, [fr1], nt)
            plsc.addupdate_scatter(dy_v, [fr2], nt)
            plsc.addupdate_scatter(dy_v, [bs + ip1], t)
            plsc.addupdate_scatter(dy_v, [bs + ip2], t)
            return carry2

        lax.fori_loop(0, BH, brow, 0)
        return carry

    lax.fori_loop(0, NG2, group2, 0)

    def group1(ig, carry):
        base = ig * 16
        ir1 = i1r1_v[pl.ds(base, 16)]
        ip1 = i1p1_v[pl.ds(base, 16)]
        ip2 = i1p2_v[pl.ds(base, 16)]
        av = a1_v[pl.ds(base, 16)]
        bv = b1_v[pl.ds(base, 16)]
        gv = g1_v[pl.ds(base, 16)]

        def brow(b, carry2):
            ltv = ltb_v[pl.ds(b * 16, 16)]
            ntv = ntb_v[pl.ds(b * 16, 16)]
            bs = jnp.full((16,), b * S, dtype=jnp.int32)
            fr1 = bs + ir1
            y1 = plsc.load_gather(y_v, [fr1])
            k = av * jnp.exp(bv * ltv + gv * ntv)
            t = k * y1
            plsc.addupdate_scatter(dy_v, [fr1], -t)
            plsc.addupdate_scatter(dy_v, [bs + ip1], t)
            plsc.addupdate_scatter(dy_v, [bs + ip2], t)
            return carry2

        lax.fori_loop(0, BH, brow, 0)
        return carry

    lax.fori_loop(0, NG1, group1, 0)

    pltpu.sync_copy(dy_v, out_h.at[s, c])


def kernel(t_in, y_in, inds_r1_1st, inds_p1_1st, inds_p2_1st,
           inds_r1_2nd, inds_r2_2nd, inds_p1_2nd, inds_p2_2nd,
           alpha_1st, beta_1st, gamma_1st,
           alpha_2nd, beta_2nd, gamma_2nd,
           T0, den0):
    T_gas = 10.0 + T0[0] * jax.nn.sigmoid(t_in * 1.0e-5)
    lt = jnp.log(T_gas / 300.0)
    ninvT = -1.0 / T_gas
    ltb = jnp.broadcast_to(lt[:, None], (B, 16)).reshape(B * 16)
    ntb = jnp.broadcast_to(ninvT[:, None], (B, 16)).reshape(B * 16)

    a2d = alpha_2nd * den0[0]

    i1r1 = _pad_groups(inds_r1_1st, R1G, R1GP, NSUB)
    i1p1 = _pad_groups(inds_p1_1st, R1G, R1GP, NSUB)
    i1p2 = _pad_groups(inds_p2_1st, R1G, R1GP, NSUB)
    a1 = _pad_groups(alpha_1st, R1G, R1GP, NSUB)
    b1 = _pad_groups(beta_1st, R1G, R1GP, NSUB)
    g1 = _pad_groups(gamma_1st, R1G, R1GP, NSUB)
    i2r1 = _pad_groups(inds_r1_2nd, R2G, R2GP, NSUB)
    i2r2 = _pad_groups(inds_r2_2nd, R2G, R2GP, NSUB)
    i2p1 = _pad_groups(inds_p1_2nd, R2G, R2GP, NSUB)
    i2p2 = _pad_groups(inds_p2_2nd, R2G, R2GP, NSUB)
    a2 = _pad_groups(a2d, R2G, R2GP, NSUB)
    b2 = _pad_groups(beta_2nd, R2G, R2GP, NSUB)
    g2 = _pad_groups(gamma_2nd, R2G, R2GP, NSUB)

    mesh = plsc.VectorSubcoreMesh(core_axis_name="c", subcore_axis_name="s")
    sc = pl.kernel(
        _sc_body,
        mesh=mesh,
        compiler_params=pltpu.CompilerParams(needs_layout_passes=False),
        out_type=jax.ShapeDtypeStruct((NSUB, NCORE, BH * S), jnp.float32),
        scratch_types=[
            pltpu.VMEM((BH * S,), jnp.float32),
            pltpu.VMEM((BH * S,), jnp.float32),
            pltpu.VMEM((BH * 16,), jnp.float32),
            pltpu.VMEM((BH * 16,), jnp.float32),
            pltpu.VMEM((R1GP,), jnp.int32),
            pltpu.VMEM((R1GP,), jnp.int32),
            pltpu.VMEM((R1GP,), jnp.int32),
            pltpu.VMEM((R1GP,), jnp.float32),
            pltpu.VMEM((R1GP,), jnp.float32),
            pltpu.VMEM((R1GP,), jnp.float32),
            pltpu.VMEM((R2GP,), jnp.int32),
            pltpu.VMEM((R2GP,), jnp.int32),
            pltpu.VMEM((R2GP,), jnp.int32),
            pltpu.VMEM((R2GP,), jnp.int32),
            pltpu.VMEM((R2GP,), jnp.float32),
            pltpu.VMEM((R2GP,), jnp.float32),
            pltpu.VMEM((R2GP,), jnp.float32),
        ],
    )
    partials = sc(y_in.reshape(B * S), ltb, ntb,
                  i1r1, i1p1, i1p2, a1, b1, g1,
                  i2r1, i2r2, i2p1, i2p2, a2, b2, g2)
    return partials.sum(axis=0).reshape(B, S)


# speedup vs baseline: 2.3781x; 1.0102x over previous
import jax
import jax.numpy as jnp
from jax import lax
from jax.experimental import pallas as pl
from jax.experimental.pallas import tpu as pltpu
from jax.experimental.pallas import tpu_sc as plsc

B = 128
S = 512
R1 = 20000
R2 = 80000

NCORE = 2
NSUB = 16
BH = B // NCORE

R1G = R1 // NSUB
R2G = R2 // NSUB
R1GP = ((R1G + 15) // 16) * 16
R2GP = ((R2G + 15) // 16) * 16
NG1 = R1GP // 16
NG2 = R2GP // 16


def _pad_groups(x, rg, rgp, nsub):
    x = x.reshape(nsub, rg)
    x = jnp.pad(x, ((0, 0), (0, rgp - rg)))
    return x.reshape(nsub * rgp)


def _sc_body(y_h, ltb_h, ntb_h,
             i1r1_h, i1p1_h, i1p2_h, a1_h, b1_h, g1_h,
             i2r1_h, i2r2_h, i2p1_h, i2p2_h, a2_h, b2_h, g2_h,
             out_h,
             y_v, dy_v, ltb_v, ntb_v,
             i1r1_v, i1p1_v, i1p2_v, a1_v, b1_v, g1_v,
             i2r1_v, i2r2_v, i2p1_v, i2p2_v, a2_v, b2_v, g2_v, dma_sem):
    c = lax.axis_index("c")
    s = lax.axis_index("s")

    copies = [
        pltpu.async_copy(y_h.at[pl.ds(c * (BH * S), BH * S)], y_v, dma_sem),
        pltpu.async_copy(ltb_h.at[pl.ds(c * (BH * 16), BH * 16)], ltb_v, dma_sem),
        pltpu.async_copy(ntb_h.at[pl.ds(c * (BH * 16), BH * 16)], ntb_v, dma_sem),
        pltpu.async_copy(i1r1_h.at[pl.ds(s * R1GP, R1GP)], i1r1_v, dma_sem),
        pltpu.async_copy(i1p1_h.at[pl.ds(s * R1GP, R1GP)], i1p1_v, dma_sem),
        pltpu.async_copy(i1p2_h.at[pl.ds(s * R1GP, R1GP)], i1p2_v, dma_sem),
        pltpu.async_copy(a1_h.at[pl.ds(s * R1GP, R1GP)], a1_v, dma_sem),
        pltpu.async_copy(b1_h.at[pl.ds(s * R1GP, R1GP)], b1_v, dma_sem),
        pltpu.async_copy(g1_h.at[pl.ds(s * R1GP, R1GP)], g1_v, dma_sem),
        pltpu.async_copy(i2r1_h.at[pl.ds(s * R2GP, R2GP)], i2r1_v, dma_sem),
        pltpu.async_copy(i2r2_h.at[pl.ds(s * R2GP, R2GP)], i2r2_v, dma_sem),
        pltpu.async_copy(i2p1_h.at[pl.ds(s * R2GP, R2GP)], i2p1_v, dma_sem),
        pltpu.async_copy(i2p2_h.at[pl.ds(s * R2GP, R2GP)], i2p2_v, dma_sem),
        pltpu.async_copy(a2_h.at[pl.ds(s * R2GP, R2GP)], a2_v, dma_sem),
        pltpu.async_copy(b2_h.at[pl.ds(s * R2GP, R2GP)], b2_v, dma_sem),
        pltpu.async_copy(g2_h.at[pl.ds(s * R2GP, R2GP)], g2_v, dma_sem),
    ]

    zeros16 = jnp.zeros((16,), jnp.float32)

    def zero_row(i, carry):
        for j in range(8):
            dy_v[pl.ds(i * 128 + j * 16, 16)] = zeros16
        return carry

    lax.fori_loop(0, BH * S // 128, zero_row, 0)

    for cp in copies:
        cp.wait()

    def group2(ig, carry):
        base = ig * 16
        ir1 = i2r1_v[pl.ds(base, 16)]
        ir2 = i2r2_v[pl.ds(base, 16)]
        ip1 = i2p1_v[pl.ds(base, 16)]
        ip2 = i2p2_v[pl.ds(base, 16)]
        av = a2_v[pl.ds(base, 16)]
        bv = b2_v[pl.ds(base, 16)]
        gv = g2_v[pl.ds(base, 16)]

        def brow(bb, carry2):
            for j in range(4):
                b = bb * 4 + j
                ltv = ltb_v[pl.ds(b * 16, 16)]
                ntv = ntb_v[pl.ds(b * 16, 16)]
                bs = jnp.full((16,), b * S, dtype=jnp.int32)
                fr1 = bs + ir1
                fr2 = bs + ir2
                y1 = plsc.load_gather(y_v, [fr1])
                y2 = plsc.load_gather(y_v, [fr2])
                k = av * jnp.exp(bv * ltv + gv * ntv)
                t = k * y1 * y2
                nt = -t
                plsc.addupdate_scatter(dy_v, [fr1], nt)
                plsc.addupdate_scatter(dy_v, [fr2], nt)
                plsc.addupdate_scatter(dy_v, [bs + ip1], t)
                plsc.addupdate_scatter(dy_v, [bs + ip2], t)
            return carry2

        lax.fori_loop(0, BH // 4, brow, 0)
        return carry

    lax.fori_loop(0, NG2, group2, 0)

    def group1(ig, carry):
        base = ig * 16
        ir1 = i1r1_v[pl.ds(base, 16)]
        ip1 = i1p1_v[pl.ds(base, 16)]
        ip2 = i1p2_v[pl.ds(base, 16)]
        av = a1_v[pl.ds(base, 16)]
        bv = b1_v[pl.ds(base, 16)]
        gv = g1_v[pl.ds(base, 16)]

        def brow(bb, carry2):
            for j in range(4):
                b = bb * 4 + j
                ltv = ltb_v[pl.ds(b * 16, 16)]
                ntv = ntb_v[pl.ds(b * 16, 16)]
                bs = jnp.full((16,), b * S, dtype=jnp.int32)
                fr1 = bs + ir1
                y1 = plsc.load_gather(y_v, [fr1])
                k = av * jnp.exp(bv * ltv + gv * ntv)
                t = k * y1
                plsc.addupdate_scatter(dy_v, [fr1], -t)
                plsc.addupdate_scatter(dy_v, [bs + ip1], t)
                plsc.addupdate_scatter(dy_v, [bs + ip2], t)
            return carry2

        lax.fori_loop(0, BH // 4, brow, 0)
        return carry

    lax.fori_loop(0, NG1, group1, 0)

    pltpu.sync_copy(dy_v, out_h.at[s, c])


def kernel(t_in, y_in, inds_r1_1st, inds_p1_1st, inds_p2_1st,
           inds_r1_2nd, inds_r2_2nd, inds_p1_2nd, inds_p2_2nd,
           alpha_1st, beta_1st, gamma_1st,
           alpha_2nd, beta_2nd, gamma_2nd,
           T0, den0):
    T_gas = 10.0 + T0[0] * jax.nn.sigmoid(t_in * 1.0e-5)
    lt = jnp.log(T_gas / 300.0)
    ninvT = -1.0 / T_gas
    ltb = jnp.broadcast_to(lt[:, None], (B, 16)).reshape(B * 16)
    ntb = jnp.broadcast_to(ninvT[:, None], (B, 16)).reshape(B * 16)

    a2d = alpha_2nd * den0[0]

    i1r1 = _pad_groups(inds_r1_1st, R1G, R1GP, NSUB)
    i1p1 = _pad_groups(inds_p1_1st, R1G, R1GP, NSUB)
    i1p2 = _pad_groups(inds_p2_1st, R1G, R1GP, NSUB)
    a1 = _pad_groups(alpha_1st, R1G, R1GP, NSUB)
    b1 = _pad_groups(beta_1st, R1G, R1GP, NSUB)
    g1 = _pad_groups(gamma_1st, R1G, R1GP, NSUB)
    i2r1 = _pad_groups(inds_r1_2nd, R2G, R2GP, NSUB)
    i2r2 = _pad_groups(inds_r2_2nd, R2G, R2GP, NSUB)
    i2p1 = _pad_groups(inds_p1_2nd, R2G, R2GP, NSUB)
    i2p2 = _pad_groups(inds_p2_2nd, R2G, R2GP, NSUB)
    a2 = _pad_groups(a2d, R2G, R2GP, NSUB)
    b2 = _pad_groups(beta_2nd, R2G, R2GP, NSUB)
    g2 = _pad_groups(gamma_2nd, R2G, R2GP, NSUB)

    mesh = plsc.VectorSubcoreMesh(core_axis_name="c", subcore_axis_name="s")
    sc = pl.kernel(
        _sc_body,
        mesh=mesh,
        compiler_params=pltpu.CompilerParams(needs_layout_passes=False),
        out_type=jax.ShapeDtypeStruct((NSUB, NCORE, BH * S), jnp.float32),
        scratch_types=[
            pltpu.VMEM((BH * S,), jnp.float32),
            pltpu.VMEM((BH * S,), jnp.float32),
            pltpu.VMEM((BH * 16,), jnp.float32),
            pltpu.VMEM((BH * 16,), jnp.float32),
            pltpu.VMEM((R1GP,), jnp.int32),
            pltpu.VMEM((R1GP,), jnp.int32),
            pltpu.VMEM((R1GP,), jnp.int32),
            pltpu.VMEM((R1GP,), jnp.float32),
            pltpu.VMEM((R1GP,), jnp.float32),
            pltpu.VMEM((R1GP,), jnp.float32),
            pltpu.VMEM((R2GP,), jnp.int32),
            pltpu.VMEM((R2GP,), jnp.int32),
            pltpu.VMEM((R2GP,), jnp.int32),
            pltpu.VMEM((R2GP,), jnp.int32),
            pltpu.VMEM((R2GP,), jnp.float32),
            pltpu.VMEM((R2GP,), jnp.float32),
            pltpu.VMEM((R2GP,), jnp.float32),
            pltpu.SemaphoreType.DMA,
        ],
    )
    partials = sc(y_in.reshape(B * S), ltb, ntb,
                  i1r1, i1p1, i1p2, a1, b1, g1,
                  i2r1, i2r2, i2p1, i2p2, a2, b2, g2)
    return partials.sum(axis=0).reshape(B, S)
